# Initial kernel scaffold; baseline (speedup 1.0000x reference)
import jax
import jax.numpy as jnp
from jax.experimental import pallas as pl


def kernel(x, emb_table, fc_w):
    raise NotImplementedError("write your pallas kernel here")


# speedup vs baseline: 1.1645x; 1.1645x over previous
import functools

import jax
import jax.numpy as jnp
from jax import lax
from jax.experimental import pallas as pl
from jax.experimental.pallas import tpu as pltpu
from jax.experimental.pallas import tpu_sc as plsc

D = 64
IDX_W = 128
CH = 1024


def _sc_gather(table, idx2d):
    n_rows_idx, _ = idx2d.shape
    n_total = n_rows_idx * IDX_W
    info = plsc.get_sparse_core_info()
    nw = info.num_cores * info.num_subcores
    per_w = n_total // nw
    n_chunks = per_w // CH
    k_streams = CH // IDX_W
    mesh = plsc.VectorSubcoreMesh(core_axis_name="c", subcore_axis_name="s")

    @functools.partial(
        pl.kernel,
        mesh=mesh,
        out_type=jax.ShapeDtypeStruct((n_total, D), jnp.float32),
        compiler_params=pltpu.CompilerParams(use_tc_tiling_on_sc=False),
        scratch_types=[
            pltpu.VMEM((k_streams, IDX_W), jnp.int32),
            pltpu.VMEM((CH, D), jnp.float32),
            pltpu.SemaphoreType.DMA,
        ],
    )
    def gather_kernel(table_hbm, idx_hbm, out_hbm, idx_v, rows_v, sem):
        wid = lax.axis_index("s") * info.num_cores + lax.axis_index("c")
        base = wid * per_w

        def body(i, carry):
            pos = pl.multiple_of(base + i * CH, CH)
            pltpu.sync_copy(
                idx_hbm.at[pl.ds(pl.multiple_of(pos // IDX_W, 8), k_streams)],
                idx_v)
            copies = [
                pltpu.async_copy(
                    table_hbm.at[idx_v.at[b]],
                    rows_v.at[pl.ds(b * IDX_W, IDX_W)],
                    sem,
                )
                for b in range(k_streams)
            ]
            for c in copies:
                c.wait()
            pltpu.sync_copy(rows_v, out_hbm.at[pl.ds(pos, CH)])
            return carry

        lax.fori_loop(0, n_chunks, body, 0)

    return gather_kernel(table, idx2d)


def _tc_matmul(e, w):
    n = e.shape[0]
    bm = 4096

    def mm(e_ref, w_ref, o_ref):
        o_ref[...] = lax.dot_general(
            e_ref[...], w_ref[...],
            (((1,), (1,)), ((), ())),
            preferred_element_type=jnp.float32,
        )

    return pl.pallas_call(
        mm,
        grid=(n // bm,),
        in_specs=[
            pl.BlockSpec((bm, D), lambda i: (i, 0)),
            pl.BlockSpec((D, D), lambda i: (0, 0)),
        ],
        out_specs=pl.BlockSpec((bm, D), lambda i: (i, 0)),
        out_shape=jax.ShapeDtypeStruct((n, D), jnp.float32),
    )(e, w)


def kernel(x, emb_table, fc_w):
    b, l = x.shape
    idx2d = x.reshape(-1, IDX_W).astype(jnp.int32)
    e = _sc_gather(emb_table, idx2d)
    out = _tc_matmul(e, fc_w)
    return out.reshape(b, l, D)


# speedup vs baseline: 1.1688x; 1.0037x over previous
import functools

import jax
import jax.numpy as jnp
from jax import lax
from jax.experimental import pallas as pl
from jax.experimental.pallas import tpu as pltpu
from jax.experimental.pallas import tpu_sc as plsc

D = 64
IDX_W = 128
GRP = 1024
CH = 512


def _sc_gather(table, idx2d):
    n_rows_idx, _ = idx2d.shape
    n_total = n_rows_idx * IDX_W
    info = plsc.get_sparse_core_info()
    nw = info.num_cores * info.num_subcores
    per_w = n_total // nw
    n_chunks = per_w // CH
    k_streams = CH // IDX_W
    grp_rows = GRP // IDX_W
    mesh = plsc.VectorSubcoreMesh(core_axis_name="c", subcore_axis_name="s")

    @functools.partial(
        pl.kernel,
        mesh=mesh,
        out_type=jax.ShapeDtypeStruct((n_total, D), jnp.float32),
        compiler_params=pltpu.CompilerParams(use_tc_tiling_on_sc=False),
        scratch_types=[
            pltpu.VMEM((grp_rows, IDX_W), jnp.int32),
            pltpu.VMEM((2, CH, D), jnp.float32),
            pltpu.SemaphoreType.DMA,
            pltpu.SemaphoreType.DMA,
        ],
    )
    def gather_kernel(table_hbm, idx_hbm, out_hbm, idx_v, rows_v, gsem, wsem):
        wid = lax.axis_index("s") * info.num_cores + lax.axis_index("c")
        base = wid * per_w

        def wb_copy(j):
            pos = pl.multiple_of(base + j * CH, CH)
            return pltpu.make_async_copy(
                rows_v.at[j % 2], out_hbm.at[pl.ds(pos, CH)], wsem)

        def body(j, carry):
            pos = pl.multiple_of(base + j * CH, CH)
            p = j % 2
            @pl.when(p == 0)
            def _():
                pltpu.sync_copy(
                    idx_hbm.at[
                        pl.ds(pl.multiple_of(pos // IDX_W, grp_rows), grp_rows)
                    ],
                    idx_v)
            @pl.when(j >= 2)
            def _():
                wb_copy(j - 2).wait()
            copies = [
                pltpu.async_copy(
                    table_hbm.at[idx_v.at[p * k_streams + b]],
                    rows_v.at[p].at[pl.ds(b * IDX_W, IDX_W)],
                    gsem,
                )
                for b in range(k_streams)
            ]
            for c in copies:
                c.wait()
            wb_copy(j).start()
            return carry

        lax.fori_loop(0, n_chunks, body, 0)
        wb_copy(n_chunks - 2).wait()
        wb_copy(n_chunks - 1).wait()

    return gather_kernel(table, idx2d)


def _tc_matmul(e, w):
    n = e.shape[0]
    bm = 4096

    def mm(e_ref, w_ref, o_ref):
        o_ref[...] = lax.dot_general(
            e_ref[...], w_ref[...],
            (((1,), (1,)), ((), ())),
            preferred_element_type=jnp.float32,
        )

    return pl.pallas_call(
        mm,
        grid=(n // bm,),
        in_specs=[
            pl.BlockSpec((bm, D), lambda i: (i, 0)),
            pl.BlockSpec((D, D), lambda i: (0, 0)),
        ],
        out_specs=pl.BlockSpec((bm, D), lambda i: (i, 0)),
        out_shape=jax.ShapeDtypeStruct((n, D), jnp.float32),
    )(e, w)


def kernel(x, emb_table, fc_w):
    b, l = x.shape
    idx2d = x.reshape(-1, IDX_W).astype(jnp.int32)
    e = _sc_gather(emb_table, idx2d)
    out = _tc_matmul(e, fc_w)
    return out.reshape(b, l, D)
